# baseline (speedup 1.0000x reference)
import functools

import jax
import jax.numpy as jnp
import numpy as np
from jax import lax
from jax.experimental import pallas as pl
from jax.experimental.pallas import tpu as pltpu
from jax.experimental.pallas import tpu_sc as plsc

B = 16384
L = 50
DIM = 32
VOCAB = 81616

NC = 2
NS = 16
NW = NC * NS

GPC = 2
CLEN = GPC * L
NCHUNK = B // GPC
CPW = NCHUNK // NW
RPW = B // NW
NBUF = 8
RPT = VOCAB // NS

_OUT_PERM = np.empty(DIM, dtype=np.int32)
_OUT_PERM[0::2] = np.arange(DIM // 2)
_OUT_PERM[1::2] = np.arange(DIM // 2, DIM)


def _sc_body(w_hbm, idx_hbm, out_hbm, table_sh, acc_v, *bufs):
    idx_bufs = bufs[0:NBUF]
    row_bufs = bufs[NBUF:2 * NBUF]
    isems, rsems = bufs[2 * NBUF], bufs[2 * NBUF + 1]

    cid = lax.axis_index("c")
    sid = lax.axis_index("s")
    wid = sid * NC + cid
    chunk0 = wid * CPW

    pltpu.sync_copy(w_hbm.at[pl.ds(sid * RPT, RPT)], table_sh.at[pl.ds(sid * RPT, RPT)])
    plsc.subcore_barrier()

    def start_idx(k, b):
        pltpu.async_copy(idx_hbm.at[chunk0 + k], idx_bufs[b], isems.at[b])

    def wait_idx(b):
        pltpu.make_async_copy(idx_hbm.at[0], idx_bufs[b], isems.at[b]).wait()

    def start_gather(b):
        pltpu.async_copy(table_sh.at[idx_bufs[b]], row_bufs[b], rsems.at[b])

    def wait_rows(b):
        pltpu.make_async_copy(table_sh.at[idx_bufs[0]], row_bufs[b], rsems.at[b]).wait()

    for b in range(NBUF):
        start_idx(b, b)
    for b in range(NBUF):
        wait_idx(b)
        start_gather(b)

    def ring_body(j, carry):
        for b in range(NBUF):
            k = NBUF * j + b
            wait_rows(b)

            @pl.when(k + NBUF < CPW)
            def _():
                start_idx(k + NBUF, b)

            for g in range(GPC):
                row0 = row_bufs[b][g * L]
                v0, v1 = plsc.unpack(
                    row0, format=plsc.PackFormat.INTERLEAVED,
                    preferred_element_type=jnp.float32)
                for r in range(1, L):
                    row = row_bufs[b][g * L + r]
                    a0, a1 = plsc.unpack(
                        row, format=plsc.PackFormat.INTERLEAVED,
                        preferred_element_type=jnp.float32)
                    v0 = v0 + a0
                    v1 = v1 + a1
                acc_v[k * GPC + g, pl.ds(0, 16)] = v0
                acc_v[k * GPC + g, pl.ds(16, 16)] = v1

            @pl.when(k + NBUF < CPW)
            def _():
                wait_idx(b)
                start_gather(b)

        return carry

    lax.fori_loop(0, CPW // NBUF, ring_body, 0)

    pltpu.sync_copy(acc_v, out_hbm.at[pl.ds(wid * RPW, RPW)])


@jax.jit
def _sc_embed_sum(w_bf16, idx_chunks):
    mesh = plsc.VectorSubcoreMesh(core_axis_name="c", subcore_axis_name="s")
    scratch = [
        pltpu.VMEM_SHARED((VOCAB, DIM), jnp.bfloat16),
        pltpu.VMEM((RPW, DIM), jnp.float32),
    ]
    scratch += [pltpu.VMEM((CLEN,), jnp.int32) for _ in range(NBUF)]
    scratch += [pltpu.VMEM((CLEN, DIM), jnp.bfloat16) for _ in range(NBUF)]
    scratch += [pltpu.SemaphoreType.DMA((NBUF,)), pltpu.SemaphoreType.DMA((NBUF,))]
    return pl.kernel(
        _sc_body,
        out_type=jax.ShapeDtypeStruct((B, DIM), jnp.float32),
        mesh=mesh,
        scratch_types=scratch,
        compiler_params=pltpu.CompilerParams(
            use_tc_tiling_on_sc=False, needs_layout_passes=False),
    )(w_bf16, idx_chunks)


def kernel(inputs, w):
    idx_chunks = inputs.astype(jnp.int32).reshape(NCHUNK, CLEN)
    w_bf16 = w.astype(jnp.bfloat16)
    out_interleaved = _sc_embed_sum(w_bf16, idx_chunks)
    return out_interleaved[:, _OUT_PERM]


# speedup vs baseline: 1.3512x; 1.3512x over previous
import functools

import jax
import jax.numpy as jnp
import numpy as np
from jax import lax
from jax.experimental import pallas as pl
from jax.experimental.pallas import tpu as pltpu
from jax.experimental.pallas import tpu_sc as plsc

B = 16384
L = 50
DIM = 32
VOCAB = 81616

NC = 2
NS = 16
NW = NC * NS

GPC = 4
CLEN = GPC * L
NCHUNK = B // GPC
CPW = NCHUNK // NW
RPW = B // NW
NBUF = 4
RPT = VOCAB // NS
G1 = 128
G2 = CLEN - G1

_OUT_PERM = np.empty(DIM, dtype=np.int32)
_OUT_PERM[0::2] = np.arange(DIM // 2)
_OUT_PERM[1::2] = np.arange(DIM // 2, DIM)


def _tree_sum(vs):
    while len(vs) > 1:
        nxt = [a + b for a, b in zip(vs[0::2], vs[1::2])]
        if len(vs) % 2:
            nxt.append(vs[-1])
        vs = nxt
    return vs[0]


def _sc_body(w_hbm, idx_hbm, out_hbm, table_sh, acc_v, *bufs):
    idx_bufs = bufs[0:NBUF]
    row_bufs = bufs[NBUF:2 * NBUF]
    isems, rsems = bufs[2 * NBUF], bufs[2 * NBUF + 1]

    cid = lax.axis_index("c")
    sid = lax.axis_index("s")
    wid = sid * NC + cid
    chunk0 = wid * CPW

    pltpu.sync_copy(w_hbm.at[pl.ds(sid * RPT, RPT)], table_sh.at[pl.ds(sid * RPT, RPT)])
    plsc.subcore_barrier()

    def start_idx(k, b):
        pltpu.async_copy(
            idx_hbm.at[pl.ds((chunk0 + k) * CLEN, CLEN)], idx_bufs[b], isems.at[b])

    def wait_idx(b):
        pltpu.make_async_copy(
            idx_hbm.at[pl.ds(0, CLEN)], idx_bufs[b], isems.at[b]).wait()

    def start_gather(b):
        pltpu.async_copy(
            table_sh.at[idx_bufs[b].at[pl.ds(0, G1)]],
            row_bufs[b].at[pl.ds(0, G1)], rsems.at[b])
        pltpu.async_copy(
            table_sh.at[idx_bufs[b].at[pl.ds(G1, G2)]],
            row_bufs[b].at[pl.ds(G1, G2)], rsems.at[b])

    def wait_rows(b):
        pltpu.make_async_copy(
            table_sh.at[idx_bufs[0]], row_bufs[b], rsems.at[b]).wait()

    for b in range(NBUF):
        start_idx(b, b)
    for b in range(NBUF):
        wait_idx(b)
        start_gather(b)

    def ring_body(j, carry):
        for b in range(NBUF):
            k = NBUF * j + b
            wait_rows(b)

            @pl.when(k + NBUF < CPW)
            def _():
                start_idx(k + NBUF, b)

            for g in range(GPC):
                s = _tree_sum([row_bufs[b][g * L + r] for r in range(L)])
                v0, v1 = plsc.unpack(
                    s, format=plsc.PackFormat.INTERLEAVED,
                    preferred_element_type=jnp.float32)
                acc_v[pl.ds((k * GPC + g) * DIM, 16)] = v0
                acc_v[pl.ds((k * GPC + g) * DIM + 16, 16)] = v1

            @pl.when(k + NBUF < CPW)
            def _():
                wait_idx(b)
                start_gather(b)

        return carry

    lax.fori_loop(0, CPW // NBUF, ring_body, 0)

    pltpu.sync_copy(acc_v, out_hbm.at[pl.ds(wid * RPW * DIM, RPW * DIM)])


@jax.jit
def _sc_embed_sum(w_bf16, idx_flat):
    mesh = plsc.VectorSubcoreMesh(core_axis_name="c", subcore_axis_name="s")
    scratch = [
        pltpu.VMEM_SHARED((VOCAB, DIM), jnp.bfloat16),
        pltpu.VMEM((RPW * DIM,), jnp.float32),
    ]
    scratch += [pltpu.VMEM((CLEN,), jnp.int32) for _ in range(NBUF)]
    scratch += [pltpu.VMEM((CLEN, DIM), jnp.bfloat16) for _ in range(NBUF)]
    scratch += [pltpu.SemaphoreType.DMA((NBUF,)), pltpu.SemaphoreType.DMA((NBUF,))]
    return pl.kernel(
        _sc_body,
        out_type=jax.ShapeDtypeStruct((B * DIM,), jnp.float32),
        mesh=mesh,
        scratch_types=scratch,
        compiler_params=pltpu.CompilerParams(
            use_tc_tiling_on_sc=False, needs_layout_passes=False),
    )(w_bf16, idx_flat)


def kernel(inputs, w):
    idx_flat = inputs.astype(jnp.int32).reshape(B * L)
    w_bf16 = w.astype(jnp.bfloat16)
    out_flat = _sc_embed_sum(w_bf16, idx_flat)
    return out_flat.reshape(B, DIM)[:, _OUT_PERM]
